# baseline (speedup 1.0000x reference)
import functools

import jax
import jax.numpy as jnp
from jax import lax
from jax.experimental import pallas as pl
from jax.experimental.pallas import tpu as pltpu
from jax.experimental.pallas import tpu_sc as plsc

NC = 2
NS = 16
NW = NC * NS
D = 32
CH = 256


def _lane_bcast(x, lane):
    dnums = lax.GatherDimensionNumbers(
        offset_dims=(), collapsed_slice_dims=(0,), start_index_map=(0,))
    return lax.gather(x, lane.reshape(16, 1), dnums, slice_sizes=(1,),
                      mode=lax.GatherScatterMode.PROMISE_IN_BOUNDS)


def _make_sc_lookup(B, L, EV, WV):
    BL = B * L
    b_per_w = B // NW
    n_half = b_per_w // CH
    mesh = plsc.VectorSubcoreMesh(core_axis_name="c", subcore_axis_name="s")

    @functools.partial(
        pl.kernel,
        mesh=mesh,
        out_type=jax.ShapeDtypeStruct((L, 2 * D, B), jnp.float32),
        compiler_params=pltpu.CompilerParams(needs_layout_passes=False),
        scratch_types=[
            pltpu.VMEM((CH,), jnp.int32),
            pltpu.VMEM((CH,), jnp.int32),
            pltpu.VMEM((CH,), jnp.int32),
            pltpu.VMEM((CH,), jnp.int32),
            pltpu.VMEM((CH, 128), jnp.float32),
            pltpu.VMEM((CH, 128), jnp.float32),
            pltpu.VMEM((D, CH + 1), jnp.float32),
            pltpu.VMEM((D, CH + 1), jnp.float32),
            pltpu.SemaphoreType.DMA,
            pltpu.SemaphoreType.DMA,
        ],
    )
    def lookup(ev_idx, wo_idx, ev_tab, wo_tab, out,
               ie_v, iw_v, re_v, rw_v, ge_v, gw_v, ve_v, vw_v, sem_e, sem_w):
        wid = lax.axis_index("s") * NC + lax.axis_index("c")
        b0 = wid * b_per_w

        def chunk(l, h):
            off = l * B + b0 + h * CH
            pltpu.sync_copy(ev_idx.at[pl.ds(off, CH)], ie_v)
            pltpu.sync_copy(wo_idx.at[pl.ds(off, CH)], iw_v)

            def blkids(j, carry):
                ie = ie_v[pl.ds(j * 16, 16)]
                iw = iw_v[pl.ds(j * 16, 16)]
                re_v[pl.ds(j * 16, 16)] = lax.shift_right_logical(ie, 2)
                rw_v[pl.ds(j * 16, 16)] = lax.shift_right_logical(iw, 2)
                return carry

            lax.fori_loop(0, CH // 16, blkids, 0)
            ce = pltpu.async_copy(ev_tab.at[re_v], ge_v, sem_e)
            cw = pltpu.async_copy(wo_tab.at[rw_v], gw_v, sem_w)
            ce.wait()
            cw.wait()

            iota = lax.iota(jnp.int32, 16)
            drows = [iota, iota + 16]

            def select(j, carry):
                ce16 = (ie_v[pl.ds(j * 16, 16)] & 3) * D
                cw16 = (iw_v[pl.ds(j * 16, 16)] & 3) * D
                for k in range(16):
                    pos = j * 16 + k
                    pos_v = jnp.full((16,), 0, jnp.int32) + pos
                    lane = jnp.full((16,), k, jnp.int32)
                    ck = _lane_bcast(ce16, lane)
                    dk = _lane_bcast(cw16, lane)
                    for hh in range(2):
                        ve = plsc.load_gather(ge_v, [pos_v, ck + drows[hh]])
                        vw = plsc.load_gather(gw_v, [pos_v, dk + drows[hh]])
                        plsc.store_scatter(ve_v, [drows[hh], pos_v], ve)
                        plsc.store_scatter(vw_v, [drows[hh], pos_v], vw)
                return carry

            lax.fori_loop(0, CH // 16, select, 0)
            pltpu.sync_copy(ve_v.at[:, pl.ds(0, CH)],
                            out.at[l, pl.ds(0, D), pl.ds(b0 + h * CH, CH)])
            pltpu.sync_copy(vw_v.at[:, pl.ds(0, CH)],
                            out.at[l, pl.ds(D, D), pl.ds(b0 + h * CH, CH)])

        def l_loop(l, carry):
            def h_loop(h, carry2):
                chunk(l, h)
                return carry2
            lax.fori_loop(0, n_half, h_loop, 0)
            return carry

        lax.fori_loop(0, L, l_loop, 0)

    return lookup


def kernel(event_ids, word_ids, event_table, word_table):
    B, L = event_ids.shape
    EV, _ = event_table.shape
    WV, _ = word_table.shape
    ev_idx = event_ids.T.reshape(B * L).astype(jnp.int32)
    wo_idx = word_ids.T.reshape(B * L).astype(jnp.int32)
    ev4 = event_table.reshape(EV // 4, 4 * D)
    wo4 = word_table.reshape(WV // 4, 4 * D)
    out = _make_sc_lookup(B, L, EV, WV)(ev_idx, wo_idx, ev4, wo4)
    return out.transpose(2, 0, 1)


# speedup vs baseline: 1.3102x; 1.3102x over previous
import functools

import jax
import jax.numpy as jnp
from jax import lax
from jax.experimental import pallas as pl
from jax.experimental.pallas import tpu as pltpu
from jax.experimental.pallas import tpu_sc as plsc

NC = 2
NS = 16
NW = NC * NS
D = 32
CH = 128


def _make_sc_lookup(B, L):
    b_per_w = B // NW
    n_h = b_per_w // CH
    n_chunks = L * n_h
    mesh = plsc.VectorSubcoreMesh(core_axis_name="c", subcore_axis_name="s")

    idx_t = pltpu.VMEM((CH,), jnp.int32)
    gbuf_t = pltpu.VMEM((CH, 128), jnp.float32)
    vals_t = pltpu.VMEM((D, CH + 1), jnp.float32)

    @functools.partial(
        pl.kernel,
        mesh=mesh,
        out_type=jax.ShapeDtypeStruct((L, 2 * D, B), jnp.float32),
        compiler_params=pltpu.CompilerParams(needs_layout_passes=False),
        scratch_types=[
            [idx_t] * 2, [idx_t] * 2,
            [idx_t] * 2, [idx_t] * 2,
            [gbuf_t] * 2, [gbuf_t] * 2,
            vals_t, vals_t,
            [pltpu.SemaphoreType.DMA] * 2,
            [pltpu.SemaphoreType.DMA] * 2,
        ],
    )
    def lookup(ev_idx, wo_idx, ev_tab, wo_tab, out,
               ie_s, iw_s, re_s, rw_s, ge_s, gw_s, ve_v, vw_v, sem_e, sem_w):
        wid = lax.axis_index("s") * NC + lax.axis_index("c")
        b0 = wid * b_per_w

        def issue(t, s):
            l = t // n_h
            h = t % n_h
            off = l * B + b0 + h * CH
            pltpu.sync_copy(ev_idx.at[pl.ds(off, CH)], ie_s[s])
            pltpu.sync_copy(wo_idx.at[pl.ds(off, CH)], iw_s[s])

            def blkids(j, carry):
                ie = ie_s[s][pl.ds(j * 16, 16)]
                iw = iw_s[s][pl.ds(j * 16, 16)]
                re_s[s][pl.ds(j * 16, 16)] = lax.shift_right_logical(ie, 2)
                rw_s[s][pl.ds(j * 16, 16)] = lax.shift_right_logical(iw, 2)
                return carry

            lax.fori_loop(0, CH // 16, blkids, 0)
            pltpu.async_copy(ev_tab.at[re_s[s]], ge_s[s], sem_e[s])
            pltpu.async_copy(wo_tab.at[rw_s[s]], gw_s[s], sem_w[s])

        def drain(t, s):
            l = t // n_h
            h = t % n_h
            pltpu.make_async_copy(ev_tab.at[re_s[s]], ge_s[s], sem_e[s]).wait()
            pltpu.make_async_copy(wo_tab.at[rw_s[s]], gw_s[s], sem_w[s]).wait()

            def select(j, carry):
                rows = lax.iota(jnp.int32, 16) + j * 16
                ce16 = (ie_s[s][pl.ds(j * 16, 16)] & 3) * D
                cw16 = (iw_s[s][pl.ds(j * 16, 16)] & 3) * D
                ev_g = [plsc.load_gather(ge_s[s], [rows, ce16 + d])
                        for d in range(D)]
                wo_g = [plsc.load_gather(gw_s[s], [rows, cw16 + d])
                        for d in range(D)]
                for d in range(D):
                    ve_v[d, pl.ds(j * 16, 16)] = ev_g[d]
                    vw_v[d, pl.ds(j * 16, 16)] = wo_g[d]
                return carry

            lax.fori_loop(0, CH // 16, select, 0)
            bc = b0 + h * CH
            pltpu.sync_copy(ve_v.at[:, pl.ds(0, CH)],
                            out.at[l, pl.ds(0, D), pl.ds(bc, CH)])
            pltpu.sync_copy(vw_v.at[:, pl.ds(0, CH)],
                            out.at[l, pl.ds(D, D), pl.ds(bc, CH)])

        issue(0, 0)

        def pipe(tt, carry):
            for s in range(2):
                t = tt * 2 + s

                @pl.when(t + 1 < n_chunks)
                def _():
                    issue(t + 1, (s + 1) % 2)

                drain(t, s)
            return carry

        lax.fori_loop(0, n_chunks // 2, pipe, 0)

    return lookup


def kernel(event_ids, word_ids, event_table, word_table):
    B, L = event_ids.shape
    EV, _ = event_table.shape
    WV, _ = word_table.shape
    ev_idx = event_ids.T.reshape(B * L).astype(jnp.int32)
    wo_idx = word_ids.T.reshape(B * L).astype(jnp.int32)
    ev4 = event_table.reshape(EV // 4, 4 * D)
    wo4 = word_table.reshape(WV // 4, 4 * D)
    out = _make_sc_lookup(B, L)(ev_idx, wo_idx, ev4, wo4)
    return out.transpose(2, 0, 1)


# speedup vs baseline: 1.3755x; 1.0499x over previous
import functools

import jax
import jax.numpy as jnp
from jax import lax
from jax.experimental import pallas as pl
from jax.experimental.pallas import tpu as pltpu
from jax.experimental.pallas import tpu_sc as plsc

NC = 2
NS = 16
NW = NC * NS
D = 32
CH = 128


def _make_sc_lookup(B, L):
    b_per_w = B // NW
    n_h = b_per_w // CH
    n_chunks = L * n_h
    mesh = plsc.VectorSubcoreMesh(core_axis_name="c", subcore_axis_name="s")

    idx_t = pltpu.VMEM((CH,), jnp.int32)
    gbuf_t = pltpu.VMEM((CH, 128), jnp.float32)
    vals_t = pltpu.VMEM((D, CH), jnp.float32)

    @functools.partial(
        pl.kernel,
        mesh=mesh,
        out_type=jax.ShapeDtypeStruct((L, 2 * D, B), jnp.float32),
        compiler_params=pltpu.CompilerParams(needs_layout_passes=False),
        scratch_types=[
            [idx_t] * 2, [idx_t] * 2,
            [idx_t] * 2, [idx_t] * 2,
            [gbuf_t] * 2, [gbuf_t] * 2,
            [vals_t] * 2, [vals_t] * 2,
            [pltpu.SemaphoreType.DMA] * 2,
            [pltpu.SemaphoreType.DMA] * 2,
            [pltpu.SemaphoreType.DMA] * 2,
            [pltpu.SemaphoreType.DMA] * 2,
        ],
    )
    def lookup(ev_idx, wo_idx, ev_tab, wo_tab, out,
               ie_s, iw_s, re_s, rw_s, ge_s, gw_s, ve_s, vw_s,
               sem_e, sem_w, sem_oe, sem_ow):
        wid = lax.axis_index("s") * NC + lax.axis_index("c")
        b0 = wid * b_per_w

        def issue(t, s):
            l = t // n_h
            h = t % n_h
            off = l * B + b0 + h * CH
            pltpu.sync_copy(ev_idx.at[pl.ds(off, CH)], ie_s[s])
            pltpu.sync_copy(wo_idx.at[pl.ds(off, CH)], iw_s[s])

            def blkids(j, carry):
                ie = ie_s[s][pl.ds(j * 16, 16)]
                iw = iw_s[s][pl.ds(j * 16, 16)]
                re_s[s][pl.ds(j * 16, 16)] = lax.shift_right_logical(ie, 2)
                rw_s[s][pl.ds(j * 16, 16)] = lax.shift_right_logical(iw, 2)
                return carry

            lax.fori_loop(0, CH // 16, blkids, 0)
            pltpu.async_copy(ev_tab.at[re_s[s]], ge_s[s], sem_e[s])
            pltpu.async_copy(wo_tab.at[rw_s[s]], gw_s[s], sem_w[s])

        def drain(t, s):
            l = t // n_h
            h = t % n_h
            pltpu.make_async_copy(ev_tab.at[re_s[s]], ge_s[s], sem_e[s]).wait()
            pltpu.make_async_copy(wo_tab.at[rw_s[s]], gw_s[s], sem_w[s]).wait()

            @pl.when(t >= 2)
            def _():
                lp = (t - 2) // n_h
                hp = (t - 2) % n_h
                bp = b0 + hp * CH
                pltpu.make_async_copy(
                    ve_s[s], out.at[lp, pl.ds(0, D), pl.ds(bp, CH)],
                    sem_oe[s]).wait()
                pltpu.make_async_copy(
                    vw_s[s], out.at[lp, pl.ds(D, D), pl.ds(bp, CH)],
                    sem_ow[s]).wait()

            def select(j, carry):
                rows = lax.iota(jnp.int32, 16) + j * 16
                ce16 = (ie_s[s][pl.ds(j * 16, 16)] & 3) * D
                cw16 = (iw_s[s][pl.ds(j * 16, 16)] & 3) * D
                ev_g = [plsc.load_gather(ge_s[s], [rows, ce16 + d])
                        for d in range(D)]
                wo_g = [plsc.load_gather(gw_s[s], [rows, cw16 + d])
                        for d in range(D)]
                for d in range(D):
                    ve_s[s][d, pl.ds(j * 16, 16)] = ev_g[d]
                    vw_s[s][d, pl.ds(j * 16, 16)] = wo_g[d]
                return carry

            lax.fori_loop(0, CH // 16, select, 0)
            bc = b0 + h * CH
            pltpu.async_copy(ve_s[s], out.at[l, pl.ds(0, D), pl.ds(bc, CH)],
                             sem_oe[s])
            pltpu.async_copy(vw_s[s], out.at[l, pl.ds(D, D), pl.ds(bc, CH)],
                             sem_ow[s])

        issue(0, 0)

        def pipe(tt, carry):
            for s in range(2):
                t = tt * 2 + s

                @pl.when(t + 1 < n_chunks)
                def _():
                    issue(t + 1, (s + 1) % 2)

                drain(t, s)
            return carry

        lax.fori_loop(0, n_chunks // 2, pipe, 0)

        for t in (n_chunks - 2, n_chunks - 1):
            s = t % 2
            l = t // n_h
            h = t % n_h
            bc = b0 + h * CH
            pltpu.make_async_copy(
                ve_s[s], out.at[l, pl.ds(0, D), pl.ds(bc, CH)],
                sem_oe[s]).wait()
            pltpu.make_async_copy(
                vw_s[s], out.at[l, pl.ds(D, D), pl.ds(bc, CH)],
                sem_ow[s]).wait()

    return lookup


def kernel(event_ids, word_ids, event_table, word_table):
    B, L = event_ids.shape
    EV, _ = event_table.shape
    WV, _ = word_table.shape
    ev_idx = event_ids.T.reshape(B * L).astype(jnp.int32)
    wo_idx = word_ids.T.reshape(B * L).astype(jnp.int32)
    ev4 = event_table.reshape(EV // 4, 4 * D)
    wo4 = word_table.reshape(WV // 4, 4 * D)
    out = _make_sc_lookup(B, L)(ev_idx, wo_idx, ev4, wo4)
    return out.transpose(2, 0, 1)


# speedup vs baseline: 1.6012x; 1.1641x over previous
import functools

import jax
import jax.numpy as jnp
from jax import lax
from jax.experimental import pallas as pl
from jax.experimental.pallas import tpu as pltpu
from jax.experimental.pallas import tpu_sc as plsc

NC = 2
NS = 16
NW = NC * NS
D = 32
CH = 128


def _make_sc_lookup(B, L):
    b_per_w = B // NW
    n_h = b_per_w // CH
    n_chunks = L * n_h
    mesh = plsc.VectorSubcoreMesh(core_axis_name="c", subcore_axis_name="s")

    idx_t = pltpu.VMEM((CH,), jnp.int32)
    gbuf_t = pltpu.VMEM((CH, 128), jnp.float32)
    vals_t = pltpu.VMEM((D, CH), jnp.float32)

    @functools.partial(
        pl.kernel,
        mesh=mesh,
        out_type=jax.ShapeDtypeStruct((L, 2 * D, B), jnp.float32),
        compiler_params=pltpu.CompilerParams(needs_layout_passes=False),
        scratch_types=[
            [idx_t] * 4, [idx_t] * 4,
            [idx_t] * 2, [idx_t] * 2,
            [gbuf_t] * 2, [gbuf_t] * 2,
            [vals_t] * 2, [vals_t] * 2,
            [pltpu.SemaphoreType.DMA] * 2,
            [pltpu.SemaphoreType.DMA] * 2,
            [pltpu.SemaphoreType.DMA] * 2,
            [pltpu.SemaphoreType.DMA] * 2,
            [pltpu.SemaphoreType.DMA] * 4,
            [pltpu.SemaphoreType.DMA] * 4,
        ],
    )
    def lookup(ev_idx, wo_idx, ev_tab, wo_tab, out,
               ie_s, iw_s, re_s, rw_s, ge_s, gw_s, ve_s, vw_s,
               sem_e, sem_w, sem_oe, sem_ow, sem_ie, sem_iw):
        wid = lax.axis_index("s") * NC + lax.axis_index("c")
        b0 = wid * b_per_w

        def stage_idx(t, si):
            l = t // n_h
            h = t % n_h
            off = l * B + b0 + h * CH
            pltpu.async_copy(ev_idx.at[pl.ds(off, CH)], ie_s[si], sem_ie[si])
            pltpu.async_copy(wo_idx.at[pl.ds(off, CH)], iw_s[si], sem_iw[si])

        def issue(t, si, s):
            l = t // n_h
            h = t % n_h
            off = l * B + b0 + h * CH
            pltpu.make_async_copy(ev_idx.at[pl.ds(off, CH)], ie_s[si],
                                  sem_ie[si]).wait()
            pltpu.make_async_copy(wo_idx.at[pl.ds(off, CH)], iw_s[si],
                                  sem_iw[si]).wait()

            def blkids(j, carry):
                ie = ie_s[si][pl.ds(j * 16, 16)]
                iw = iw_s[si][pl.ds(j * 16, 16)]
                re_s[s][pl.ds(j * 16, 16)] = lax.shift_right_logical(ie, 2)
                rw_s[s][pl.ds(j * 16, 16)] = lax.shift_right_logical(iw, 2)
                return carry

            lax.fori_loop(0, CH // 16, blkids, 0)
            pltpu.async_copy(ev_tab.at[re_s[s]], ge_s[s], sem_e[s])
            pltpu.async_copy(wo_tab.at[rw_s[s]], gw_s[s], sem_w[s])

        def drain(t, si, s):
            l = t // n_h
            h = t % n_h
            pltpu.make_async_copy(ev_tab.at[re_s[s]], ge_s[s], sem_e[s]).wait()
            pltpu.make_async_copy(wo_tab.at[rw_s[s]], gw_s[s], sem_w[s]).wait()

            @pl.when(t >= 2)
            def _():
                lp = (t - 2) // n_h
                hp = (t - 2) % n_h
                bp = b0 + hp * CH
                pltpu.make_async_copy(
                    ve_s[s], out.at[lp, pl.ds(0, D), pl.ds(bp, CH)],
                    sem_oe[s]).wait()
                pltpu.make_async_copy(
                    vw_s[s], out.at[lp, pl.ds(D, D), pl.ds(bp, CH)],
                    sem_ow[s]).wait()

            def select(j, carry):
                rows = lax.iota(jnp.int32, 16) + j * 16
                ce16 = (ie_s[si][pl.ds(j * 16, 16)] & 3) * D
                cw16 = (iw_s[si][pl.ds(j * 16, 16)] & 3) * D
                ev_g = [plsc.load_gather(ge_s[s], [rows, ce16 + d])
                        for d in range(D)]
                wo_g = [plsc.load_gather(gw_s[s], [rows, cw16 + d])
                        for d in range(D)]
                for d in range(D):
                    ve_s[s][d, pl.ds(j * 16, 16)] = ev_g[d]
                    vw_s[s][d, pl.ds(j * 16, 16)] = wo_g[d]
                return carry

            lax.fori_loop(0, CH // 16, select, 0)
            bc = b0 + h * CH
            pltpu.async_copy(ve_s[s], out.at[l, pl.ds(0, D), pl.ds(bc, CH)],
                             sem_oe[s])
            pltpu.async_copy(vw_s[s], out.at[l, pl.ds(D, D), pl.ds(bc, CH)],
                             sem_ow[s])

        stage_idx(0, 0)
        stage_idx(1, 1)
        issue(0, 0, 0)

        def pipe(tt, carry):
            for s in range(4):
                t = tt * 4 + s

                @pl.when(t + 2 < n_chunks)
                def _():
                    stage_idx(t + 2, (s + 2) % 4)

                @pl.when(t + 1 < n_chunks)
                def _():
                    issue(t + 1, (s + 1) % 4, (s + 1) % 2)

                drain(t, s, s % 2)
            return carry

        lax.fori_loop(0, n_chunks // 4, pipe, 0)

        for t in (n_chunks - 2, n_chunks - 1):
            s = t % 2
            l = t // n_h
            h = t % n_h
            bc = b0 + h * CH
            pltpu.make_async_copy(
                ve_s[s], out.at[l, pl.ds(0, D), pl.ds(bc, CH)],
                sem_oe[s]).wait()
            pltpu.make_async_copy(
                vw_s[s], out.at[l, pl.ds(D, D), pl.ds(bc, CH)],
                sem_ow[s]).wait()

    return lookup


def kernel(event_ids, word_ids, event_table, word_table):
    B, L = event_ids.shape
    EV, _ = event_table.shape
    WV, _ = word_table.shape
    ev_idx = event_ids.T.reshape(B * L).astype(jnp.int32)
    wo_idx = word_ids.T.reshape(B * L).astype(jnp.int32)
    ev4 = event_table.reshape(EV // 4, 4 * D)
    wo4 = word_table.reshape(WV // 4, 4 * D)
    out = _make_sc_lookup(B, L)(ev_idx, wo_idx, ev4, wo4)
    return out.transpose(2, 0, 1)
